# Initial kernel scaffold; baseline (speedup 1.0000x reference)
import jax
import jax.numpy as jnp
from jax.experimental import pallas as pl


def kernel(table, a_idx, b_idx, neg_idx):
    raise NotImplementedError("write your pallas kernel here")


# speedup vs baseline: 1.8247x; 1.8247x over previous
import functools

import jax
import jax.numpy as jnp
from jax import lax
from jax.experimental import pallas as pl
from jax.experimental.pallas import tpu as pltpu
from jax.experimental.pallas import tpu_sc as plsc

D = 64
DOC = 50
CD = 8
RC = CD * DOC
NW = 32
MAXN2 = 400.0


def _sc_pool(table, idx_flat, n_docs):
    docs_per_w = n_docs // NW
    chunks = docs_per_w // CD
    mesh = plsc.VectorSubcoreMesh(core_axis_name="c", subcore_axis_name="s")

    @functools.partial(
        pl.kernel,
        out_type=jax.ShapeDtypeStruct((n_docs, D), jnp.float32),
        mesh=mesh,
        scratch_types=[
            pltpu.VMEM((RC,), jnp.int32),
            pltpu.VMEM((RC, D), jnp.float32),
            pltpu.VMEM((CD, D), jnp.float32),
            pltpu.SemaphoreType.DMA,
        ],
        compiler_params=pltpu.CompilerParams(
            needs_layout_passes=False, use_tc_tiling_on_sc=False),
    )
    def k(table_hbm, idx_hbm, out_hbm, idx_v, rows_v, out_v, sem):
        wid = lax.axis_index("c") * 16 + lax.axis_index("s")
        base_doc = wid * docs_per_w

        def chunk_body(c, carry):
            doc0 = base_doc + c * CD
            pltpu.sync_copy(idx_hbm.at[pl.ds(doc0 * DOC, RC)], idx_v)
            pltpu.async_copy(table_hbm.at[idx_v], rows_v, sem).wait()
            for d in range(CD):
                def tok_body(t, acc):
                    r = d * DOC + t
                    v = [rows_v[r, pl.ds(j * 16, 16)] for j in range(4)]
                    ssq = plsc.cumsum(v[0] * v[0] + v[1] * v[1]
                                      + v[2] * v[2] + v[3] * v[3])[15]
                    xi = lax.bitcast_convert_type(ssq, jnp.int32)
                    y = lax.bitcast_convert_type(
                        jnp.int32(0x5F3759DF) - (xi >> 1), jnp.float32)
                    y = y * (1.5 - 0.5 * ssq * y * y)
                    y = y * (1.5 - 0.5 * ssq * y * y)
                    y = y * (1.5 - 0.5 * ssq * y * y)
                    scale = jnp.where(ssq > MAXN2, 20.0 * y,
                                      jnp.float32(1.0))
                    return tuple(a + vv * scale for a, vv in zip(acc, v))

                zero = jnp.zeros((16,), jnp.float32)
                acc = lax.fori_loop(0, DOC, tok_body, (zero,) * 4)
                for j in range(4):
                    out_v[d, pl.ds(j * 16, 16)] = acc[j]
            pltpu.sync_copy(out_v, out_hbm.at[pl.ds(doc0, CD)])
            return carry

        lax.fori_loop(0, chunks, chunk_body, 0)

    return k(table, idx_flat)


def kernel(table, a_idx, b_idx, neg_idx):
    b = a_idx.shape[0]
    idx_flat = jnp.concatenate([
        a_idx.reshape(-1), b_idx.reshape(-1), neg_idx.reshape(-1),
    ]).astype(jnp.int32)
    out = _sc_pool(table, idx_flat, 5 * b)
    l_batch = out[:b][:, None, :]
    r_batch = out[b:2 * b][:, None, :]
    neg_batch = out[2 * b:].reshape(b, neg_idx.shape[1], D)
    return (l_batch, r_batch, neg_batch)


# speedup vs baseline: 2.6300x; 1.4413x over previous
import functools

import jax
import jax.numpy as jnp
from jax import lax
from jax.experimental import pallas as pl
from jax.experimental.pallas import tpu as pltpu
from jax.experimental.pallas import tpu_sc as plsc

D = 64
DOC = 50
CD = 8
RC = CD * DOC
NW = 32
MAXN2 = 400.0


def _row_scale(ssq):
    xi = lax.bitcast_convert_type(ssq, jnp.int32)
    y = lax.bitcast_convert_type(
        jnp.int32(0x5F3759DF) - (xi >> 1), jnp.float32)
    y = y * (1.5 - 0.5 * ssq * y * y)
    y = y * (1.5 - 0.5 * ssq * y * y)
    y = y * (1.5 - 0.5 * ssq * y * y)
    return jnp.where(ssq > MAXN2, 20.0 * y, jnp.float32(1.0))


def _sc_pool(table, idx_flat, n_docs):
    docs_per_w = n_docs // NW
    chunks = docs_per_w // CD
    pairs = chunks // 2
    mesh = plsc.VectorSubcoreMesh(core_axis_name="c", subcore_axis_name="s")

    @functools.partial(
        pl.kernel,
        out_type=jax.ShapeDtypeStruct((n_docs, D), jnp.float32),
        mesh=mesh,
        scratch_types=[
            pltpu.VMEM((docs_per_w * DOC,), jnp.int32),
            pltpu.VMEM((RC, D), jnp.float32),
            pltpu.VMEM((RC, D), jnp.float32),
            pltpu.VMEM((docs_per_w, D), jnp.float32),
            pltpu.SemaphoreType.DMA,
            pltpu.SemaphoreType.DMA,
        ],
        compiler_params=pltpu.CompilerParams(
            needs_layout_passes=False, use_tc_tiling_on_sc=False),
    )
    def k(table_hbm, idx_hbm, out_hbm, idx_v, rows0, rows1, out_v,
          sem0, sem1):
        wid = lax.axis_index("c") * 16 + lax.axis_index("s")
        base_doc = wid * docs_per_w

        pltpu.sync_copy(idx_hbm.at[pl.ds(base_doc * DOC, docs_per_w * DOC)],
                        idx_v)

        def gather_start(c, rows, sem):
            pltpu.async_copy(table_hbm.at[idx_v.at[pl.ds(c * RC, RC)]],
                             rows, sem)

        def gather_wait(rows, sem):
            pltpu.make_async_copy(table_hbm.at[idx_v.at[pl.ds(0, RC)]],
                                  rows, sem).wait()

        zero = jnp.zeros((16,), jnp.float32)

        def compute(rows_v, c):
            gm = (zero,) * 4
            for d in range(CD):
                def tok_fast(t, carry):
                    r = d * DOC + t
                    v = [rows_v[r, pl.ds(j * 16, 16)] for j in range(4)]
                    acc = tuple(a + vv for a, vv in zip(carry[:4], v))
                    g = tuple(jnp.maximum(gg, vv * vv)
                              for gg, vv in zip(carry[4:], v))
                    return acc + g

                res = lax.fori_loop(0, DOC, tok_fast, (zero,) * 4 + gm,
                                    unroll=2)
                for j in range(4):
                    out_v[c * CD + d, pl.ds(j * 16, 16)] = res[j]
                gm = res[4:]

            g = jnp.maximum(jnp.maximum(gm[0], gm[1]),
                            jnp.maximum(gm[2], gm[3]))
            bound = plsc.cummax(g)[15] * jnp.float32(D)

            @pl.when(bound > MAXN2)
            def _slow():
                for d in range(CD):
                    def tok_slow(t, acc):
                        r = d * DOC + t
                        v = [rows_v[r, pl.ds(j * 16, 16)] for j in range(4)]
                        ssq = plsc.cumsum(v[0] * v[0] + v[1] * v[1]
                                          + v[2] * v[2] + v[3] * v[3])[15]
                        s = _row_scale(ssq)
                        return tuple(a + vv * s for a, vv in zip(acc, v))

                    acc = lax.fori_loop(0, DOC, tok_slow, (zero,) * 4)
                    for j in range(4):
                        out_v[c * CD + d, pl.ds(j * 16, 16)] = acc[j]

        gather_start(0, rows0, sem0)

        def pair_body(i, carry):
            c0 = 2 * i
            gather_start(c0 + 1, rows1, sem1)
            gather_wait(rows0, sem0)
            compute(rows0, c0)

            @pl.when(c0 + 2 < chunks)
            def _():
                gather_start(c0 + 2, rows0, sem0)

            gather_wait(rows1, sem1)
            compute(rows1, c0 + 1)
            return carry

        lax.fori_loop(0, pairs, pair_body, 0)
        pltpu.sync_copy(out_v, out_hbm.at[pl.ds(base_doc, docs_per_w)])

    return k(table, idx_flat)


def kernel(table, a_idx, b_idx, neg_idx):
    b = a_idx.shape[0]
    idx_flat = jnp.concatenate([
        a_idx.reshape(-1), b_idx.reshape(-1), neg_idx.reshape(-1),
    ]).astype(jnp.int32)
    out = _sc_pool(table, idx_flat, 5 * b)
    l_batch = out[:b][:, None, :]
    r_batch = out[b:2 * b][:, None, :]
    neg_batch = out[2 * b:].reshape(b, neg_idx.shape[1], D)
    return (l_batch, r_batch, neg_batch)
